# baseline (speedup 1.0000x reference)
import functools

import jax
import jax.numpy as jnp
from jax.experimental import pallas as pl
from jax.experimental.pallas import tpu as pltpu

H = 1024
F = 512
E = 8
FS = 1024
T = 2048


def _gate_body(x_ref, gwt_ref, bias_ref, cmb_ref):
    x = x_ref[...]
    logits = jnp.dot(x, gwt_ref[...], preferred_element_type=jnp.float32)
    scores = jax.nn.sigmoid(logits)
    sfc = scores + bias_ref[...]
    col = jax.lax.broadcasted_iota(jnp.int32, (T, E), 1)
    m1 = jnp.max(sfc, axis=1, keepdims=True)
    i1 = jnp.min(jnp.where(sfc == m1, col, E), axis=1, keepdims=True)
    oh1 = (col == i1).astype(jnp.float32)
    sfc2 = jnp.where(oh1 > 0, -jnp.inf, sfc)
    m2 = jnp.max(sfc2, axis=1, keepdims=True)
    i2 = jnp.min(jnp.where(sfc2 == m2, col, E), axis=1, keepdims=True)
    oh2 = (col == i2).astype(jnp.float32)
    s1 = jnp.sum(oh1 * scores, axis=1, keepdims=True)
    s2 = jnp.sum(oh2 * scores, axis=1, keepdims=True)
    denom = s1 + s2 + 1e-20
    cmb_ref[...] = oh1 * (s1 / denom) + oh2 * (s2 / denom)


def _gate(x, gate_w, gate_bias):
    return pl.pallas_call(
        _gate_body,
        out_shape=jax.ShapeDtypeStruct((T, E), jnp.float32),
    )(x, gate_w.T, gate_bias.reshape(1, E))


def _silu(v):
    return v * jax.nn.sigmoid(v)


def _shared_body(x_ref, sg_ref, su_ref, sd_ref, o_ref):
    x = x_ref[...]
    g = jnp.dot(x, sg_ref[...], preferred_element_type=jnp.float32)
    u = jnp.dot(x, su_ref[...], preferred_element_type=jnp.float32)
    p = (_silu(g) * u).astype(jnp.bfloat16)
    o_ref[...] = jnp.dot(p, sd_ref[...], preferred_element_type=jnp.float32)


def _shared(x, sg, su, sd):
    return pl.pallas_call(
        _shared_body,
        out_shape=jax.ShapeDtypeStruct((T, H), jnp.float32),
    )(x, sg, su, sd)


def _moe_body(x_ref, cmb_ref, sh_ref, w1_ref, w2_ref, w3_ref, o_ref):
    e = pl.program_id(0)
    x = x_ref[...]
    h1 = jnp.dot(x, w1_ref[0], preferred_element_type=jnp.float32)
    h3 = jnp.dot(x, w3_ref[0], preferred_element_type=jnp.float32)
    p = (_silu(h1) * h3).astype(jnp.bfloat16)
    y = jnp.dot(p, w2_ref[0], preferred_element_type=jnp.float32)
    cmb = cmb_ref[...]
    col = jax.lax.broadcasted_iota(jnp.int32, (T, E), 1)
    ce = jnp.sum(jnp.where(col == e, cmb, 0.0), axis=1, keepdims=True)
    y = ce * y

    @pl.when(e == 0)
    def _():
        o_ref[...] = y + sh_ref[...]

    @pl.when(e != 0)
    def _():
        o_ref[...] += y


def _moe(x, combine, shared_out, w1, w2, w3):
    return pl.pallas_call(
        _moe_body,
        grid=(E,),
        in_specs=[
            pl.BlockSpec((T, H), lambda e: (0, 0)),
            pl.BlockSpec((T, E), lambda e: (0, 0)),
            pl.BlockSpec((T, H), lambda e: (0, 0)),
            pl.BlockSpec((1, H, F), lambda e: (e, 0, 0)),
            pl.BlockSpec((1, F, H), lambda e: (e, 0, 0)),
            pl.BlockSpec((1, H, F), lambda e: (e, 0, 0)),
        ],
        out_specs=pl.BlockSpec((T, H), lambda e: (0, 0)),
        out_shape=jax.ShapeDtypeStruct((T, H), jnp.float32),
        compiler_params=pltpu.CompilerParams(
            dimension_semantics=("arbitrary",),
        ),
    )(x, combine, shared_out, w1, w2, w3)


@jax.jit
def kernel(hidden_states, gate_w, gate_bias, w1, w2, w3, sg, su, sd):
    orig_shape = hidden_states.shape
    x = hidden_states.reshape(T, H)
    combine = _gate(x, gate_w, gate_bias)
    xb = x.astype(jnp.bfloat16)
    sh = _shared(xb, sg.astype(jnp.bfloat16), su.astype(jnp.bfloat16),
                 sd.astype(jnp.bfloat16))
    out = _moe(xb, combine, sh, w1.astype(jnp.bfloat16),
               w2.astype(jnp.bfloat16), w3.astype(jnp.bfloat16))
    return out.reshape(orig_shape)


# speedup vs baseline: 1.2709x; 1.2709x over previous
import functools

import jax
import jax.numpy as jnp
from jax.experimental import pallas as pl
from jax.experimental.pallas import tpu as pltpu

H = 1024
F = 512
E = 8
FS = 1024
T = 2048


def _gate_body(x_ref, gwt_ref, bias_ref, cmb_ref):
    x = x_ref[...]
    logits = jnp.dot(x, gwt_ref[...], preferred_element_type=jnp.float32)
    scores = jax.nn.sigmoid(logits)
    sfc = scores + bias_ref[...]
    col = jax.lax.broadcasted_iota(jnp.int32, (T, E), 1)
    m1 = jnp.max(sfc, axis=1, keepdims=True)
    i1 = jnp.min(jnp.where(sfc == m1, col, E), axis=1, keepdims=True)
    oh1 = (col == i1).astype(jnp.float32)
    sfc2 = jnp.where(oh1 > 0, -jnp.inf, sfc)
    m2 = jnp.max(sfc2, axis=1, keepdims=True)
    i2 = jnp.min(jnp.where(sfc2 == m2, col, E), axis=1, keepdims=True)
    oh2 = (col == i2).astype(jnp.float32)
    s1 = jnp.sum(oh1 * scores, axis=1, keepdims=True)
    s2 = jnp.sum(oh2 * scores, axis=1, keepdims=True)
    denom = s1 + s2 + 1e-20
    cmb_ref[...] = oh1 * (s1 / denom) + oh2 * (s2 / denom)


def _gate(x, gate_w, gate_bias):
    return pl.pallas_call(
        _gate_body,
        out_shape=jax.ShapeDtypeStruct((T, E), jnp.float32),
    )(x, gate_w.T, gate_bias.reshape(1, E))


def _silu(v):
    return v * jax.nn.sigmoid(v)


def _shared_body(x_ref, sg_ref, su_ref, sd_ref, o_ref):
    x = x_ref[...].astype(jnp.bfloat16)
    g = jnp.dot(x, sg_ref[...].astype(jnp.bfloat16),
                preferred_element_type=jnp.float32)
    u = jnp.dot(x, su_ref[...].astype(jnp.bfloat16),
                preferred_element_type=jnp.float32)
    p = (_silu(g) * u).astype(jnp.bfloat16)
    o_ref[...] = jnp.dot(p, sd_ref[...].astype(jnp.bfloat16),
                         preferred_element_type=jnp.float32)


def _shared(x, sg, su, sd):
    return pl.pallas_call(
        _shared_body,
        out_shape=jax.ShapeDtypeStruct((T, H), jnp.float32),
    )(x, sg, su, sd)


def _moe_body(x_ref, cmb_ref, sh_ref, w1_ref, w2_ref, w3_ref, o_ref):
    e = pl.program_id(0)
    x = x_ref[...].astype(jnp.bfloat16)
    h1 = jnp.dot(x, w1_ref[0].astype(jnp.bfloat16),
                 preferred_element_type=jnp.float32)
    h3 = jnp.dot(x, w3_ref[0].astype(jnp.bfloat16),
                 preferred_element_type=jnp.float32)
    p = (_silu(h1) * h3).astype(jnp.bfloat16)
    y = jnp.dot(p, w2_ref[0].astype(jnp.bfloat16),
                preferred_element_type=jnp.float32)
    cmb = cmb_ref[...]
    col = jax.lax.broadcasted_iota(jnp.int32, (T, E), 1)
    ce = jnp.sum(jnp.where(col == e, cmb, 0.0), axis=1, keepdims=True)
    y = ce * y

    @pl.when(e == 0)
    def _():
        o_ref[...] = y + sh_ref[...]

    @pl.when(e != 0)
    def _():
        o_ref[...] += y


def _moe(x, combine, shared_out, w1, w2, w3):
    return pl.pallas_call(
        _moe_body,
        grid=(E,),
        in_specs=[
            pl.BlockSpec((T, H), lambda e: (0, 0)),
            pl.BlockSpec((T, E), lambda e: (0, 0)),
            pl.BlockSpec((T, H), lambda e: (0, 0)),
            pl.BlockSpec((1, H, F), lambda e: (e, 0, 0)),
            pl.BlockSpec((1, F, H), lambda e: (e, 0, 0)),
            pl.BlockSpec((1, H, F), lambda e: (e, 0, 0)),
        ],
        out_specs=pl.BlockSpec((T, H), lambda e: (0, 0)),
        out_shape=jax.ShapeDtypeStruct((T, H), jnp.float32),
        compiler_params=pltpu.CompilerParams(
            dimension_semantics=("arbitrary",),
        ),
    )(x, combine, shared_out, w1, w2, w3)


@jax.jit
def kernel(hidden_states, gate_w, gate_bias, w1, w2, w3, sg, su, sd):
    orig_shape = hidden_states.shape
    x = hidden_states.reshape(T, H)
    combine = _gate(x, gate_w, gate_bias)
    sh = _shared(x, sg, su, sd)
    out = _moe(x, combine, sh, w1, w2, w3)
    return out.reshape(orig_shape)
